# baseline (speedup 1.0000x reference)
import functools

import jax
import jax.numpy as jnp
from jax import lax
from jax.experimental import pallas as pl
from jax.experimental.pallas import tpu as pltpu
from jax.experimental.pallas import tpu_sc as plsc

N = 10000
E = 320000
D = 128
G = 64
C = 10

NC = 2
NS = 16
NW = NC * NS

NPAD = 10240
K = 64
CH = 160
NSTG = 4
CHH = CH // NSTG
NBUF = 4
EPT = K * CH
EPAD = NW * EPT
ROWS_PT = NPAD // NS

BLK = 10240
GRID = NPAD // BLK


def _deg_body(dst_hbm, degp_hbm, dst_v, ones_v, z_v, deg_sh):
    c = lax.axis_index("c")
    s = lax.axis_index("s")
    wid = s * NC + c

    pltpu.sync_copy(dst_hbm.at[wid], dst_v)

    def _fill_ones(i, carry):
        ones_v[pl.ds(i * 16, 16)] = jnp.ones((16,), jnp.float32)
        return carry

    lax.fori_loop(0, K // 16, _fill_ones, 0)

    def _fill_z(i, carry):
        z_v[pl.ds(i * 16, 16)] = jnp.zeros((16,), jnp.float32)
        return carry

    lax.fori_loop(0, ROWS_PT // 16, _fill_z, 0)

    pltpu.sync_copy(z_v, deg_sh.at[pl.ds(s * ROWS_PT, ROWS_PT)])
    plsc.subcore_barrier()

    def _scatter(ch, carry):
        pltpu.sync_copy(ones_v, deg_sh.at[dst_v.at[ch]], add=True)
        return carry

    lax.fori_loop(0, CH, _scatter, 0)
    plsc.subcore_barrier()
    pltpu.sync_copy(deg_sh.at[pl.ds(s * ROWS_PT, ROWS_PT)],
                    degp_hbm.at[c, pl.ds(s * ROWS_PT, ROWS_PT)])


def _edge_body(h_hbm, src_hbm, dst_hbm, accp_hbm,
               src_v, dst_v, rows_v, acc_sh, *sems):
    c = lax.axis_index("c")
    s = lax.axis_index("s")
    wid = s * NC + c

    pltpu.sync_copy(src_hbm.at[wid, pl.ds(0, CHH)], src_v)
    pltpu.sync_copy(dst_hbm.at[wid, pl.ds(0, CHH)], dst_v)
    for b in range(NBUF - 1):
        pltpu.async_copy(h_hbm.at[src_v.at[b]], rows_v.at[b], sems[b])

    def _fill_z(r, carry):
        for j in range(D // 16):
            rows_v[NBUF - 1, r, pl.ds(j * 16, 16)] = jnp.zeros(
                (16,), jnp.float32)
        return carry

    lax.fori_loop(0, K, _fill_z, 0)
    for t in range(ROWS_PT // K):
        pltpu.sync_copy(rows_v.at[NBUF - 1],
                        acc_sh.at[pl.ds(s * ROWS_PT + t * K, K)])
    plsc.subcore_barrier()

    for half in range(NSTG):
        if half > 0:
            pltpu.sync_copy(src_hbm.at[wid, pl.ds(half * CHH, CHH)], src_v)
            pltpu.sync_copy(dst_hbm.at[wid, pl.ds(half * CHH, CHH)], dst_v)
            for b in range(NBUF - 1):
                pltpu.async_copy(h_hbm.at[src_v.at[b]], rows_v.at[b],
                                 sems[b])

        def _body(i, carry):
            for b in range(NBUF):
                ch = NBUF * i + b
                nxt = ch + NBUF - 1

                @pl.when(nxt < CHH)
                def _():
                    nb = (b + NBUF - 1) % NBUF
                    pltpu.async_copy(h_hbm.at[src_v.at[nxt]], rows_v.at[nb],
                                     sems[nb])

                pltpu.make_async_copy(h_hbm.at[src_v.at[ch]], rows_v.at[b],
                                      sems[b]).wait()
                pltpu.sync_copy(rows_v.at[b], acc_sh.at[dst_v.at[ch]],
                                add=True)
            return carry

        lax.fori_loop(0, CHH // NBUF, _body, 0)
    plsc.subcore_barrier()
    pltpu.sync_copy(acc_sh.at[pl.ds(s * ROWS_PT, ROWS_PT)],
                    accp_hbm.at[c, pl.ds(s * ROWS_PT, ROWS_PT)])


@functools.lru_cache(maxsize=None)
def _sc_kernels():
    mesh = plsc.VectorSubcoreMesh(core_axis_name="c", subcore_axis_name="s",
                                  num_cores=NC, num_subcores=NS)
    deg_k = pl.kernel(
        _deg_body,
        out_type=jax.ShapeDtypeStruct((NC, NPAD), jnp.float32),
        mesh=mesh,
        scratch_types=[
            pltpu.VMEM((CH, K), jnp.int32),
            pltpu.VMEM((K,), jnp.float32),
            pltpu.VMEM((ROWS_PT,), jnp.float32),
            pltpu.VMEM_SHARED((NPAD,), jnp.float32),
        ],
    )
    edge_k = pl.kernel(
        _edge_body,
        out_type=jax.ShapeDtypeStruct((NC, NPAD, D), jnp.float32),
        mesh=mesh,
        scratch_types=[
            pltpu.VMEM((CHH, K), jnp.int32),
            pltpu.VMEM((CHH, K), jnp.int32),
            pltpu.VMEM((NBUF, K, D), jnp.float32),
            pltpu.VMEM_SHARED((NPAD, D), jnp.float32),
        ] + [pltpu.SemaphoreType.DMA] * NBUF,
    )
    return deg_k, edge_k


def _scale_body(x_ref, w_ref, degp_ref, dinv_ref, h_ref):
    deg = degp_ref[0, :] + degp_ref[1, :] + 1.0
    dinv = lax.rsqrt(deg)
    dinv_ref[...] = dinv
    h_ref[...] = dinv[:, None] * jnp.dot(
        x_ref[...], w_ref[...], preferred_element_type=jnp.float32)


_scale = pl.pallas_call(
    _scale_body,
    grid=(GRID,),
    in_specs=[
        pl.BlockSpec((BLK, D), lambda i: (i, 0)),
        pl.BlockSpec((D, D), lambda i: (0, 0)),
        pl.BlockSpec((2, BLK), lambda i: (0, i)),
    ],
    out_specs=[
        pl.BlockSpec((BLK,), lambda i: (i,)),
        pl.BlockSpec((BLK, D), lambda i: (i, 0)),
    ],
    out_shape=[
        jax.ShapeDtypeStruct((NPAD,), jnp.float32),
        jax.ShapeDtypeStruct((NPAD, D), jnp.float32),
    ],
)


def _finish_rows(accp, h, dinv, b):
    out = dinv[:, None] * (accp[0] + accp[1] + h) + b[None, :]
    nrm = jnp.sqrt(jnp.sum(out * out, axis=1, keepdims=True))
    r = out / jnp.maximum(nrm, 1e-12)
    return jnp.maximum(r, 0.0)


def _combine_body(accp_ref, h_ref, dinv_ref, b_ref, w_ref, hn_ref):
    r = _finish_rows(accp_ref[...], h_ref[...], dinv_ref[...], b_ref[...])
    hn_ref[...] = dinv_ref[...][:, None] * jnp.dot(
        r, w_ref[...], preferred_element_type=jnp.float32)


_combine = pl.pallas_call(
    _combine_body,
    grid=(GRID,),
    in_specs=[
        pl.BlockSpec((2, BLK, D), lambda i: (0, i, 0)),
        pl.BlockSpec((BLK, D), lambda i: (i, 0)),
        pl.BlockSpec((BLK,), lambda i: (i,)),
        pl.BlockSpec((D,), lambda i: (0,)),
        pl.BlockSpec((D, D), lambda i: (0, 0)),
    ],
    out_specs=pl.BlockSpec((BLK, D), lambda i: (i, 0)),
    out_shape=jax.ShapeDtypeStruct((NPAD, D), jnp.float32),
)


def _last_pool_body(accp_ref, h_ref, dinv_ref, b_ref, batch_ref, wl_ref,
                    bl_ref, out_ref, sums, cnts):
    i = pl.program_id(0)

    @pl.when(i == 0)
    def _():
        sums[...] = jnp.zeros_like(sums)
        cnts[...] = jnp.zeros_like(cnts)

    r = _finish_rows(accp_ref[...], h_ref[...], dinv_ref[...], b_ref[...])
    onehot = (lax.broadcasted_iota(jnp.int32, (G, BLK), 0)
              == batch_ref[...][None, :]).astype(jnp.float32)
    sums[...] += jnp.dot(onehot, r, preferred_element_type=jnp.float32)
    cnts[...] = cnts[...] + jnp.sum(onehot, axis=1, keepdims=True)

    @pl.when(i == GRID - 1)
    def _():
        mean = sums[...] / jnp.maximum(cnts[...], 1.0)
        out_ref[...] = jnp.dot(mean, wl_ref[...],
                               preferred_element_type=jnp.float32) \
            + bl_ref[...][None, :]


_last_pool = pl.pallas_call(
    _last_pool_body,
    grid=(GRID,),
    in_specs=[
        pl.BlockSpec((2, BLK, D), lambda i: (0, i, 0)),
        pl.BlockSpec((BLK, D), lambda i: (i, 0)),
        pl.BlockSpec((BLK,), lambda i: (i,)),
        pl.BlockSpec((D,), lambda i: (0,)),
        pl.BlockSpec((BLK,), lambda i: (i,)),
        pl.BlockSpec((D, C), lambda i: (0, 0)),
        pl.BlockSpec((C,), lambda i: (0,)),
    ],
    out_specs=pl.BlockSpec((G, C), lambda i: (0, 0)),
    out_shape=jax.ShapeDtypeStruct((G, C), jnp.float32),
    scratch_shapes=[
        pltpu.VMEM((G, D), jnp.float32),
        pltpu.VMEM((G, D), jnp.float32),
    ],
)


def kernel(x, edge_index, batch, W1, b1, W2, b2, W3, b3, Wl, bl):
    pad_i = jnp.arange(EPAD - E, dtype=jnp.int32)
    src = jnp.concatenate(
        [edge_index[0], pad_i % N]).reshape(NW, CH, K)
    dst = jnp.concatenate(
        [edge_index[1], N + pad_i % (NPAD - N)]).reshape(NW, CH, K)
    x_p = jnp.pad(x, ((0, NPAD - N), (0, 0)))
    batch_p = jnp.pad(batch, (0, NPAD - N), constant_values=G)

    deg_k, edge_k = _sc_kernels()
    degp = deg_k(dst)
    dinv, h1 = _scale(x_p, W1, degp)
    acc1 = edge_k(h1, src, dst)
    h2 = _combine(acc1, h1, dinv, b1, W2)
    acc2 = edge_k(h2, src, dst)
    h3 = _combine(acc2, h2, dinv, b2, W3)
    acc3 = edge_k(h3, src, dst)
    return _last_pool(acc3, h3, dinv, b3, batch_p, Wl, bl)


# speedup vs baseline: 1.0215x; 1.0215x over previous
import functools

import jax
import jax.numpy as jnp
from jax import lax
from jax.experimental import pallas as pl
from jax.experimental.pallas import tpu as pltpu
from jax.experimental.pallas import tpu_sc as plsc

N = 10000
E = 320000
D = 128
G = 64
C = 10

NC = 2
NS = 16
NW = NC * NS

NPAD = 10240
K = 64
CH = 160
NSTG = 4
CHH = CH // NSTG
NBUF = 4
EPT = K * CH
EPAD = NW * EPT
ROWS_PT = NPAD // NS

BLK = 5120
GRID = NPAD // BLK


def _deg_body(dst_hbm, degp_hbm, dst_v, ones_v, z_v, deg_sh):
    c = lax.axis_index("c")
    s = lax.axis_index("s")
    wid = s * NC + c

    pltpu.sync_copy(dst_hbm.at[wid], dst_v)

    def _fill_ones(i, carry):
        ones_v[pl.ds(i * 16, 16)] = jnp.ones((16,), jnp.float32)
        return carry

    lax.fori_loop(0, K // 16, _fill_ones, 0)

    def _fill_z(i, carry):
        z_v[pl.ds(i * 16, 16)] = jnp.zeros((16,), jnp.float32)
        return carry

    lax.fori_loop(0, ROWS_PT // 16, _fill_z, 0)

    pltpu.sync_copy(z_v, deg_sh.at[pl.ds(s * ROWS_PT, ROWS_PT)])
    plsc.subcore_barrier()

    def _scatter(ch, carry):
        pltpu.sync_copy(ones_v, deg_sh.at[dst_v.at[ch]], add=True)
        return carry

    lax.fori_loop(0, CH, _scatter, 0)
    plsc.subcore_barrier()
    pltpu.sync_copy(deg_sh.at[pl.ds(s * ROWS_PT, ROWS_PT)],
                    degp_hbm.at[c, pl.ds(s * ROWS_PT, ROWS_PT)])


def _edge_body(h_hbm, src_hbm, dst_hbm, accp_hbm,
               src_v, dst_v, rows_v, acc_sh, *sems):
    c = lax.axis_index("c")
    s = lax.axis_index("s")
    wid = s * NC + c

    pltpu.sync_copy(src_hbm.at[wid, pl.ds(0, CHH)], src_v)
    pltpu.sync_copy(dst_hbm.at[wid, pl.ds(0, CHH)], dst_v)
    for b in range(NBUF - 1):
        pltpu.async_copy(h_hbm.at[src_v.at[b]], rows_v.at[b], sems[b])

    def _fill_z(r, carry):
        for j in range(D // 16):
            rows_v[NBUF - 1, r, pl.ds(j * 16, 16)] = jnp.zeros(
                (16,), jnp.float32)
        return carry

    lax.fori_loop(0, K, _fill_z, 0)
    for t in range(ROWS_PT // K):
        pltpu.sync_copy(rows_v.at[NBUF - 1],
                        acc_sh.at[pl.ds(s * ROWS_PT + t * K, K)])
    plsc.subcore_barrier()

    for half in range(NSTG):
        if half > 0:
            pltpu.sync_copy(src_hbm.at[wid, pl.ds(half * CHH, CHH)], src_v)
            pltpu.sync_copy(dst_hbm.at[wid, pl.ds(half * CHH, CHH)], dst_v)
            for b in range(NBUF - 1):
                pltpu.async_copy(h_hbm.at[src_v.at[b]], rows_v.at[b],
                                 sems[b])

        def _body(i, carry):
            for b in range(NBUF):
                ch = NBUF * i + b
                nxt = ch + NBUF - 1

                @pl.when(nxt < CHH)
                def _():
                    nb = (b + NBUF - 1) % NBUF
                    pltpu.async_copy(h_hbm.at[src_v.at[nxt]], rows_v.at[nb],
                                     sems[nb])

                pltpu.make_async_copy(h_hbm.at[src_v.at[ch]], rows_v.at[b],
                                      sems[b]).wait()
                pltpu.sync_copy(rows_v.at[b], acc_sh.at[dst_v.at[ch]],
                                add=True)
            return carry

        lax.fori_loop(0, CHH // NBUF, _body, 0)
    plsc.subcore_barrier()
    pltpu.sync_copy(acc_sh.at[pl.ds(s * ROWS_PT, ROWS_PT)],
                    accp_hbm.at[c, pl.ds(s * ROWS_PT, ROWS_PT)])


@functools.lru_cache(maxsize=None)
def _sc_kernels():
    mesh = plsc.VectorSubcoreMesh(core_axis_name="c", subcore_axis_name="s",
                                  num_cores=NC, num_subcores=NS)
    deg_k = pl.kernel(
        _deg_body,
        out_type=jax.ShapeDtypeStruct((NC, NPAD), jnp.float32),
        mesh=mesh,
        scratch_types=[
            pltpu.VMEM((CH, K), jnp.int32),
            pltpu.VMEM((K,), jnp.float32),
            pltpu.VMEM((ROWS_PT,), jnp.float32),
            pltpu.VMEM_SHARED((NPAD,), jnp.float32),
        ],
    )
    edge_k = pl.kernel(
        _edge_body,
        out_type=jax.ShapeDtypeStruct((NC, NPAD, D), jnp.float32),
        mesh=mesh,
        scratch_types=[
            pltpu.VMEM((CHH, K), jnp.int32),
            pltpu.VMEM((CHH, K), jnp.int32),
            pltpu.VMEM((NBUF, K, D), jnp.float32),
            pltpu.VMEM_SHARED((NPAD, D), jnp.float32),
        ] + [pltpu.SemaphoreType.DMA] * NBUF,
    )
    return deg_k, edge_k


def _scale_body(x_ref, w_ref, degp_ref, dinv_ref, h_ref):
    deg = degp_ref[0, :] + degp_ref[1, :] + 1.0
    dinv = lax.rsqrt(deg)
    dinv_ref[...] = dinv
    h_ref[...] = dinv[:, None] * jnp.dot(
        x_ref[...], w_ref[...], preferred_element_type=jnp.float32)


_scale = pl.pallas_call(
    _scale_body,
    grid=(GRID,),
    in_specs=[
        pl.BlockSpec((BLK, D), lambda i: (i, 0)),
        pl.BlockSpec((D, D), lambda i: (0, 0)),
        pl.BlockSpec((2, BLK), lambda i: (0, i)),
    ],
    out_specs=[
        pl.BlockSpec((BLK,), lambda i: (i,)),
        pl.BlockSpec((BLK, D), lambda i: (i, 0)),
    ],
    out_shape=[
        jax.ShapeDtypeStruct((NPAD,), jnp.float32),
        jax.ShapeDtypeStruct((NPAD, D), jnp.float32),
    ],
)


def _finish_rows(accp, h, dinv, b):
    out = dinv[:, None] * (accp[0] + accp[1] + h) + b[None, :]
    nrm = jnp.sqrt(jnp.sum(out * out, axis=1, keepdims=True))
    r = out / jnp.maximum(nrm, 1e-12)
    return jnp.maximum(r, 0.0)


def _combine_body(accp_ref, h_ref, dinv_ref, b_ref, w_ref, hn_ref):
    r = _finish_rows(accp_ref[...], h_ref[...], dinv_ref[...], b_ref[...])
    hn_ref[...] = dinv_ref[...][:, None] * jnp.dot(
        r, w_ref[...], preferred_element_type=jnp.float32)


_combine = pl.pallas_call(
    _combine_body,
    grid=(GRID,),
    in_specs=[
        pl.BlockSpec((2, BLK, D), lambda i: (0, i, 0)),
        pl.BlockSpec((BLK, D), lambda i: (i, 0)),
        pl.BlockSpec((BLK,), lambda i: (i,)),
        pl.BlockSpec((D,), lambda i: (0,)),
        pl.BlockSpec((D, D), lambda i: (0, 0)),
    ],
    out_specs=pl.BlockSpec((BLK, D), lambda i: (i, 0)),
    out_shape=jax.ShapeDtypeStruct((NPAD, D), jnp.float32),
)


def _last_pool_body(accp_ref, h_ref, dinv_ref, b_ref, batch_ref, wl_ref,
                    bl_ref, out_ref, sums, cnts):
    i = pl.program_id(0)

    @pl.when(i == 0)
    def _():
        sums[...] = jnp.zeros_like(sums)
        cnts[...] = jnp.zeros_like(cnts)

    r = _finish_rows(accp_ref[...], h_ref[...], dinv_ref[...], b_ref[...])
    onehot = (lax.broadcasted_iota(jnp.int32, (G, BLK), 0)
              == batch_ref[...][None, :]).astype(jnp.float32)
    sums[...] += jnp.dot(onehot, r, preferred_element_type=jnp.float32)
    cnts[...] = cnts[...] + jnp.sum(onehot, axis=1, keepdims=True)

    @pl.when(i == GRID - 1)
    def _():
        mean = sums[...] / jnp.maximum(cnts[...], 1.0)
        out_ref[...] = jnp.dot(mean, wl_ref[...],
                               preferred_element_type=jnp.float32) \
            + bl_ref[...][None, :]


_last_pool = pl.pallas_call(
    _last_pool_body,
    grid=(GRID,),
    in_specs=[
        pl.BlockSpec((2, BLK, D), lambda i: (0, i, 0)),
        pl.BlockSpec((BLK, D), lambda i: (i, 0)),
        pl.BlockSpec((BLK,), lambda i: (i,)),
        pl.BlockSpec((D,), lambda i: (0,)),
        pl.BlockSpec((BLK,), lambda i: (i,)),
        pl.BlockSpec((D, C), lambda i: (0, 0)),
        pl.BlockSpec((C,), lambda i: (0,)),
    ],
    out_specs=pl.BlockSpec((G, C), lambda i: (0, 0)),
    out_shape=jax.ShapeDtypeStruct((G, C), jnp.float32),
    scratch_shapes=[
        pltpu.VMEM((G, D), jnp.float32),
        pltpu.VMEM((G, D), jnp.float32),
    ],
)


def kernel(x, edge_index, batch, W1, b1, W2, b2, W3, b3, Wl, bl):
    pad_i = jnp.arange(EPAD - E, dtype=jnp.int32)
    src = jnp.concatenate(
        [edge_index[0], pad_i % N]).reshape(NW, CH, K)
    dst = jnp.concatenate(
        [edge_index[1], N + pad_i % (NPAD - N)]).reshape(NW, CH, K)
    x_p = jnp.pad(x, ((0, NPAD - N), (0, 0)))
    batch_p = jnp.pad(batch, (0, NPAD - N), constant_values=G)

    deg_k, edge_k = _sc_kernels()
    degp = deg_k(dst)
    dinv, h1 = _scale(x_p, W1, degp)
    acc1 = edge_k(h1, src, dst)
    h2 = _combine(acc1, h1, dinv, b1, W2)
    acc2 = edge_k(h2, src, dst)
    h3 = _combine(acc2, h2, dinv, b2, W3)
    acc3 = edge_k(h3, src, dst)
    return _last_pool(acc3, h3, dinv, b3, batch_p, Wl, bl)
